# Initial kernel scaffold; baseline (speedup 1.0000x reference)
import functools

import jax
import jax.numpy as jnp
from jax import lax
from jax.experimental import pallas as pl
from jax.experimental.pallas import tpu as pltpu
from jax.experimental.pallas import tpu_sc as plsc

NC, NS, LANES = 2, 16, 16
NW = NC * NS

_MESH = plsc.VectorSubcoreMesh(core_axis_name="c", subcore_axis_name="s")

f32 = jnp.float32


def _node_mm_body(h_ref, wa, ba, wb, bb, wd, bd, wem, bem,
                  ah_ref, bhlo_ref, bhhi_ref, dh_ref, eh_ref):
    hb = h_ref[...]
    ah_ref[...] = jnp.dot(hb, wa[...], preferred_element_type=f32) + ba[...]
    bh = jnp.dot(hb, wb[...], preferred_element_type=f32) + bb[...]
    bhlo_ref[...] = bh[:, :128]
    bhhi_ref[...] = bh[:, 128:]
    dh_ref[...] = jnp.dot(hb, wd[...], preferred_element_type=f32) + bd[...]
    eh_ref[...] = jnp.dot(hb, wem[...], preferred_element_type=f32) + bem[...]


def _node_mms(h, WA, bA, WB, bB, WD, bD, WEm, bEm):
    N, D = h.shape
    BN = 2000
    grid = (N // BN,)
    row_spec = pl.BlockSpec((BN, D), lambda i: (i, 0))
    w_spec = pl.BlockSpec((D, D), lambda i: (0, 0))
    b_spec = pl.BlockSpec((1, D), lambda i: (0, 0))
    half_spec = pl.BlockSpec((BN, D // 2), lambda i: (i, 0))
    return pl.pallas_call(
        _node_mm_body,
        grid=grid,
        in_specs=[row_spec, w_spec, b_spec, w_spec, b_spec, w_spec, b_spec,
                  w_spec, b_spec],
        out_specs=[row_spec, half_spec, half_spec, row_spec, row_spec],
        out_shape=[
            jax.ShapeDtypeStruct((N, D), f32),
            jax.ShapeDtypeStruct((N, D // 2), f32),
            jax.ShapeDtypeStruct((N, D // 2), f32),
            jax.ShapeDtypeStruct((N, D), f32),
            jax.ShapeDtypeStruct((N, D), f32),
        ],
    )(h, WA, bA.reshape(1, D), WB, bB.reshape(1, D), WD, bD.reshape(1, D),
      WEm, bEm.reshape(1, D))


def _edge_mm_body(e_ref, wc, bc, ce_ref):
    ce_ref[...] = jnp.dot(e_ref[...], wc[...], preferred_element_type=f32) + bc[...]


def _edge_mm(e, WC, bC):
    E, D = e.shape
    BE = 2000
    return pl.pallas_call(
        _edge_mm_body,
        grid=(E // BE,),
        in_specs=[pl.BlockSpec((BE, D), lambda i: (i, 0)),
                  pl.BlockSpec((D, D), lambda i: (0, 0)),
                  pl.BlockSpec((1, D), lambda i: (0, 0))],
        out_specs=pl.BlockSpec((BE, D), lambda i: (i, 0)),
        out_shape=jax.ShapeDtypeStruct((E, D), f32),
    )(e, WC, bC.reshape(1, D))


def _sc_gather_body(src_hbm, dst_hbm, dh_hbm, eh_hbm, ce_hbm,
                    lo_hbm, hi_hbm,
                    sidx, didx, gbuf, ebuf, cbuf, lobuf, hibuf, s0, s1, s2,
                    *, nchunks, per, CH):
    cid = lax.axis_index("c")
    sid = lax.axis_index("s")
    wid = sid * NC + cid

    def chunk_body(j, carry):
        chunk = wid + j * NW

        @pl.when(chunk < nchunks)
        def _():
            base = chunk * CH
            pltpu.sync_copy(src_hbm.at[pl.ds(base, CH)], sidx)
            pltpu.sync_copy(dst_hbm.at[pl.ds(base, CH)], didx)
            c0 = pltpu.async_copy(dh_hbm.at[sidx], gbuf, s0)
            c1 = pltpu.async_copy(eh_hbm.at[didx], ebuf, s1)
            c2 = pltpu.async_copy(ce_hbm.at[pl.ds(base, CH)], cbuf, s2)
            c0.wait()
            c1.wait()
            c2.wait()

            def row(r, cc):
                for q in range(8):
                    sl = pl.ds(q * 16, 16)
                    sh = pl.ds(q * 16 + 128, 16)
                    lobuf[r, sl] = gbuf[r, sl] + ebuf[r, sl] + cbuf[r, sl]
                    hibuf[r, sl] = gbuf[r, sh] + ebuf[r, sh] + cbuf[r, sh]
                return cc

            lax.fori_loop(0, CH, row, 0)
            pltpu.sync_copy(lobuf, lo_hbm.at[pl.ds(base, CH)])
            pltpu.sync_copy(hibuf, hi_hbm.at[pl.ds(base, CH)])

        return carry

    lax.fori_loop(0, per, chunk_body, 0)


def _sc_gather(src, dst, dh, eh, ce):
    E = src.shape[0]
    N, D = dh.shape
    CH = 64
    nchunks = E // CH
    per = (nchunks + NW - 1) // NW
    gk = pl.kernel(
        functools.partial(_sc_gather_body, nchunks=nchunks, per=per, CH=CH),
        out_type=(jax.ShapeDtypeStruct((E, D // 2), f32),
                  jax.ShapeDtypeStruct((E, D // 2), f32)),
        mesh=_MESH,
        scratch_types=[
            pltpu.VMEM((CH,), jnp.int32),
            pltpu.VMEM((CH,), jnp.int32),
            pltpu.VMEM((CH, D), f32),
            pltpu.VMEM((CH, D), f32),
            pltpu.VMEM((CH, D), f32),
            pltpu.VMEM((CH, D // 2), f32),
            pltpu.VMEM((CH, D // 2), f32),
            pltpu.SemaphoreType.DMA,
            pltpu.SemaphoreType.DMA,
            pltpu.SemaphoreType.DMA,
        ],
    )
    return gk(src, dst, dh, eh, ce)


def _sc_scatter_body(dst_hbm, src_hbm, eplo_hbm, ephi_hbm, bhlo_hbm, bhhi_hbm,
                     siglo_hbm, sighi_hbm, shlo_hbm, shhi_hbm,
                     didx, sidx, xbuf, sbuf, bbuf, zbuf, sem, acc,
                     *, N, E, CH):
    cid = lax.axis_index("c")
    sid = lax.axis_index("s")
    H = 128
    nchunks = E // CH
    per = (nchunks + NS - 1) // NS
    rows_pt = N // NS
    zq = 5
    zrows = rows_pt // zq

    def zrow(r, cc):
        for q in range(H // 16):
            zbuf[r, pl.ds(q * 16, 16)] = jnp.zeros((16,), f32)
        return cc

    lax.fori_loop(0, zrows, zrow, 0)

    def zero_acc():
        for q in range(zq):
            pltpu.sync_copy(zbuf, acc.at[pl.ds(sid * rows_pt + q * zrows, zrows)])

    def phase(ep_ref, bh_ref, mul_b):
        def chunk_body(j, carry):
            chunk = sid + j * NS

            @pl.when(chunk < nchunks)
            def _():
                base = chunk * CH
                pltpu.sync_copy(dst_hbm.at[pl.ds(base, CH)], didx)
                pltpu.sync_copy(ep_ref.at[pl.ds(base, CH)], xbuf)
                if mul_b:
                    pltpu.sync_copy(src_hbm.at[pl.ds(base, CH)], sidx)
                    pltpu.async_copy(bh_ref.at[sidx], bbuf, sem).wait()

                def row(r, cc):
                    for q in range(H // 16):
                        sl = pl.ds(q * 16, 16)
                        x = xbuf[r, sl]
                        s = 1.0 / (1.0 + jnp.exp(-x))
                        if mul_b:
                            s = s * bbuf[r, sl]
                        sbuf[r, sl] = s
                    return cc

                lax.fori_loop(0, CH, row, 0)
                pltpu.sync_copy(sbuf, acc.at[didx], add=True)

            return carry

        lax.fori_loop(0, per, chunk_body, 0)

    def dump(out_ref):
        pltpu.sync_copy(acc.at[pl.ds(sid * rows_pt, rows_pt)],
                        out_ref.at[pl.ds(sid * rows_pt, rows_pt)])

    def run(ep_ref, bh_ref, sig_ref, sh_ref):
        zero_acc()
        plsc.subcore_barrier()
        phase(ep_ref, bh_ref, False)
        plsc.subcore_barrier()
        dump(sig_ref)
        zero_acc()
        plsc.subcore_barrier()
        phase(ep_ref, bh_ref, True)
        plsc.subcore_barrier()
        dump(sh_ref)

    @pl.when(cid == 0)
    def _():
        run(eplo_hbm, bhlo_hbm, siglo_hbm, shlo_hbm)

    @pl.when(cid == 1)
    def _():
        run(ephi_hbm, bhhi_hbm, sighi_hbm, shhi_hbm)


def _sc_scatter(dst, src, ep_lo, ep_hi, bh_lo, bh_hi):
    E = dst.shape[0]
    N = bh_lo.shape[0]
    H = 128
    CH = 128
    sk = pl.kernel(
        functools.partial(_sc_scatter_body, N=N, E=E, CH=CH),
        out_type=(jax.ShapeDtypeStruct((N, H), f32),
                  jax.ShapeDtypeStruct((N, H), f32),
                  jax.ShapeDtypeStruct((N, H), f32),
                  jax.ShapeDtypeStruct((N, H), f32)),
        mesh=_MESH,
        scratch_types=[
            pltpu.VMEM((CH,), jnp.int32),
            pltpu.VMEM((CH,), jnp.int32),
            pltpu.VMEM((CH, H), f32),
            pltpu.VMEM((CH, H), f32),
            pltpu.VMEM((CH, H), f32),
            pltpu.VMEM((N // NS // 5, H), f32),
            pltpu.SemaphoreType.DMA,
            pltpu.VMEM_SHARED((N, H), f32),
        ],
    )
    return sk(dst, src, ep_lo, ep_hi, bh_lo, bh_hi)


def _fin_h_a_body(ah_ref, slo_ref, shi_ref, hlo_ref, hhi_ref, hnew_ref, st_ref):
    i = pl.program_id(0)
    ssig = jnp.concatenate([slo_ref[...], shi_ref[...]], axis=1)
    ssh = jnp.concatenate([hlo_ref[...], hhi_ref[...]], axis=1)
    hn = ah_ref[...] + ssh / (ssig + 1e-6)
    hnew_ref[...] = hn
    s1 = jnp.sum(hn, axis=0, keepdims=True)
    s2 = jnp.sum(hn * hn, axis=0, keepdims=True)
    blk = jnp.concatenate([s1, s2], axis=0)

    @pl.when(i == 0)
    def _():
        st_ref[...] = blk

    @pl.when(i > 0)
    def _():
        st_ref[...] = st_ref[...] + blk


def _fin_h_a(ah, ssig_lo, ssig_hi, ssh_lo, ssh_hi):
    N, D = ah.shape
    BN = 2000
    row = pl.BlockSpec((BN, D), lambda i: (i, 0))
    half = pl.BlockSpec((BN, D // 2), lambda i: (i, 0))
    return pl.pallas_call(
        _fin_h_a_body,
        grid=(N // BN,),
        in_specs=[row, half, half, half, half],
        out_specs=[row, pl.BlockSpec((2, D), lambda i: (0, 0))],
        out_shape=[jax.ShapeDtypeStruct((N, D), f32),
                   jax.ShapeDtypeStruct((2, D), f32)],
    )(ah, ssig_lo, ssig_hi, ssh_lo, ssh_hi)


def _fin_norm_body(x_ref, xn_ref, st_ref, g_ref, b_ref, out_ref, *, count):
    mean = st_ref[0:1, :] / count
    var = st_ref[1:2, :] / count - mean * mean
    inv = lax.rsqrt(var + 1e-5)
    xn = (xn_ref[...] - mean) * inv * g_ref[...] + b_ref[...]
    out_ref[...] = x_ref[...] + jnp.maximum(xn, 0.0)


def _fin_h_b(h, h_new, st, gamma, beta):
    N, D = h.shape
    BN = 2000
    row = pl.BlockSpec((BN, D), lambda i: (i, 0))
    return pl.pallas_call(
        functools.partial(_fin_norm_body, count=float(N)),
        grid=(N // BN,),
        in_specs=[row, row, pl.BlockSpec((2, D), lambda i: (0, 0)),
                  pl.BlockSpec((1, D), lambda i: (0, 0)),
                  pl.BlockSpec((1, D), lambda i: (0, 0))],
        out_specs=row,
        out_shape=jax.ShapeDtypeStruct((N, D), f32),
    )(h, h_new, st, gamma.reshape(1, D), beta.reshape(1, D))


def _fin_e_a_body(lo_ref, hi_ref, st_ref):
    i = pl.program_id(0)
    x = jnp.concatenate([lo_ref[...], hi_ref[...]], axis=1)
    s1 = jnp.sum(x, axis=0, keepdims=True)
    s2 = jnp.sum(x * x, axis=0, keepdims=True)
    blk = jnp.concatenate([s1, s2], axis=0)

    @pl.when(i == 0)
    def _():
        st_ref[...] = blk

    @pl.when(i > 0)
    def _():
        st_ref[...] = st_ref[...] + blk


def _fin_e_a(ep_lo, ep_hi):
    E, H = ep_lo.shape
    D = 2 * H
    BE = 2000
    half = pl.BlockSpec((BE, H), lambda i: (i, 0))
    return pl.pallas_call(
        _fin_e_a_body,
        grid=(E // BE,),
        in_specs=[half, half],
        out_specs=pl.BlockSpec((2, D), lambda i: (0, 0)),
        out_shape=jax.ShapeDtypeStruct((2, D), f32),
    )(ep_lo, ep_hi)


def _fin_e_b_body(e_ref, lo_ref, hi_ref, st_ref, g_ref, b_ref, out_ref, *, count):
    mean = st_ref[0:1, :] / count
    var = st_ref[1:2, :] / count - mean * mean
    inv = lax.rsqrt(var + 1e-5)
    x = jnp.concatenate([lo_ref[...], hi_ref[...]], axis=1)
    xn = (x - mean) * inv * g_ref[...] + b_ref[...]
    out_ref[...] = e_ref[...] + jnp.maximum(xn, 0.0)


def _fin_e_b(e, ep_lo, ep_hi, st, gamma, beta):
    E, D = e.shape
    BE = 2000
    row = pl.BlockSpec((BE, D), lambda i: (i, 0))
    half = pl.BlockSpec((BE, D // 2), lambda i: (i, 0))
    return pl.pallas_call(
        functools.partial(_fin_e_b_body, count=float(E)),
        grid=(E // BE,),
        in_specs=[row, half, half, pl.BlockSpec((2, D), lambda i: (0, 0)),
                  pl.BlockSpec((1, D), lambda i: (0, 0)),
                  pl.BlockSpec((1, D), lambda i: (0, 0))],
        out_specs=row,
        out_shape=jax.ShapeDtypeStruct((E, D), f32),
    )(e, ep_lo, ep_hi, st, gamma.reshape(1, D), beta.reshape(1, D))


def kernel(h, e, edge_index, WA, bA, WB, bB, WC, bC, WD, bD, WEm, bEm,
           bn_gh, bn_bh, bn_ge, bn_be):
    src = edge_index[0]
    dst = edge_index[1]
    ah, bh_lo, bh_hi, dh, eh = _node_mms(h, WA, bA, WB, bB, WD, bD, WEm, bEm)
    ce = _edge_mm(e, WC, bC)
    ep_lo, ep_hi = _sc_gather(src, dst, dh, eh, ce)
    ssig_lo, ssig_hi, ssh_lo, ssh_hi = _sc_scatter(dst, src, ep_lo, ep_hi,
                                                   bh_lo, bh_hi)
    h_new, st_h = _fin_h_a(ah, ssig_lo, ssig_hi, ssh_lo, ssh_hi)
    h_out = _fin_h_b(h, h_new, st_h, bn_gh, bn_bh)
    st_e = _fin_e_a(ep_lo, ep_hi)
    e_out = _fin_e_b(e, ep_lo, ep_hi, st_e, bn_ge, bn_be)
    return (h_out, e_out)


# speedup vs baseline: 1.4955x; 1.4955x over previous
import functools

import jax
import jax.numpy as jnp
from jax import lax
from jax.experimental import pallas as pl
from jax.experimental.pallas import tpu as pltpu
from jax.experimental.pallas import tpu_sc as plsc

NC, NS, LANES = 2, 16, 16
NW = NC * NS

_MESH = plsc.VectorSubcoreMesh(core_axis_name="c", subcore_axis_name="s")

f32 = jnp.float32


def _node_mm_body(h_ref, wa, ba, wb, bb, wd, bd, wem, bem,
                  ah_ref, bhlo_ref, bhhi_ref, dh_ref, eh_ref):
    hb = h_ref[...]
    ah_ref[...] = jnp.dot(hb, wa[...], preferred_element_type=f32) + ba[...]
    bh = jnp.dot(hb, wb[...], preferred_element_type=f32) + bb[...]
    bhlo_ref[...] = bh[:, :128]
    bhhi_ref[...] = bh[:, 128:]
    dh_ref[...] = jnp.dot(hb, wd[...], preferred_element_type=f32) + bd[...]
    eh_ref[...] = jnp.dot(hb, wem[...], preferred_element_type=f32) + bem[...]


def _node_mms(h, WA, bA, WB, bB, WD, bD, WEm, bEm):
    N, D = h.shape
    BN = 2000
    grid = (N // BN,)
    row_spec = pl.BlockSpec((BN, D), lambda i: (i, 0))
    w_spec = pl.BlockSpec((D, D), lambda i: (0, 0))
    b_spec = pl.BlockSpec((1, D), lambda i: (0, 0))
    half_spec = pl.BlockSpec((BN, D // 2), lambda i: (i, 0))
    return pl.pallas_call(
        _node_mm_body,
        grid=grid,
        in_specs=[row_spec, w_spec, b_spec, w_spec, b_spec, w_spec, b_spec,
                  w_spec, b_spec],
        out_specs=[row_spec, half_spec, half_spec, row_spec, row_spec],
        out_shape=[
            jax.ShapeDtypeStruct((N, D), f32),
            jax.ShapeDtypeStruct((N, D // 2), f32),
            jax.ShapeDtypeStruct((N, D // 2), f32),
            jax.ShapeDtypeStruct((N, D), f32),
            jax.ShapeDtypeStruct((N, D), f32),
        ],
    )(h, WA, bA.reshape(1, D), WB, bB.reshape(1, D), WD, bD.reshape(1, D),
      WEm, bEm.reshape(1, D))


def _edge_mm_body(e_ref, wc, bc, ce_ref):
    ce_ref[...] = jnp.dot(e_ref[...], wc[...], preferred_element_type=f32) + bc[...]


def _edge_mm(e, WC, bC):
    E, D = e.shape
    BE = 2000
    return pl.pallas_call(
        _edge_mm_body,
        grid=(E // BE,),
        in_specs=[pl.BlockSpec((BE, D), lambda i: (i, 0)),
                  pl.BlockSpec((D, D), lambda i: (0, 0)),
                  pl.BlockSpec((1, D), lambda i: (0, 0))],
        out_specs=pl.BlockSpec((BE, D), lambda i: (i, 0)),
        out_shape=jax.ShapeDtypeStruct((E, D), f32),
    )(e, WC, bC.reshape(1, D))


def _sc_gather_body(src_hbm, dst_hbm, dh_hbm, eh_hbm, ce_hbm,
                    lo_hbm, hi_hbm,
                    sidx, didx, gbuf, ebuf, cbuf, lobuf, hibuf, s0, s1, s2,
                    *, nchunks, per, CH):
    cid = lax.axis_index("c")
    sid = lax.axis_index("s")
    wid = sid * NC + cid

    def chunk_body(j, carry):
        chunk = wid + j * NW

        @pl.when(chunk < nchunks)
        def _():
            base = chunk * CH
            pltpu.sync_copy(src_hbm.at[pl.ds(base, CH)], sidx)
            pltpu.sync_copy(dst_hbm.at[pl.ds(base, CH)], didx)
            c0 = pltpu.async_copy(dh_hbm.at[sidx], gbuf, s0)
            c1 = pltpu.async_copy(eh_hbm.at[didx], ebuf, s1)
            c2 = pltpu.async_copy(ce_hbm.at[pl.ds(base, CH)], cbuf, s2)
            c0.wait()
            c1.wait()
            c2.wait()

            def row(r, cc):
                for q in range(8):
                    sl = pl.ds(q * 16, 16)
                    sh = pl.ds(q * 16 + 128, 16)
                    lobuf[r, sl] = gbuf[r, sl] + ebuf[r, sl] + cbuf[r, sl]
                    hibuf[r, sl] = gbuf[r, sh] + ebuf[r, sh] + cbuf[r, sh]
                return cc

            lax.fori_loop(0, CH, row, 0)
            pltpu.sync_copy(lobuf, lo_hbm.at[pl.ds(base, CH)])
            pltpu.sync_copy(hibuf, hi_hbm.at[pl.ds(base, CH)])

        return carry

    lax.fori_loop(0, per, chunk_body, 0)


def _sc_gather(src, dst, dh, eh, ce):
    E = src.shape[0]
    N, D = dh.shape
    CH = 64
    nchunks = E // CH
    per = (nchunks + NW - 1) // NW
    gk = pl.kernel(
        functools.partial(_sc_gather_body, nchunks=nchunks, per=per, CH=CH),
        out_type=(jax.ShapeDtypeStruct((E, D // 2), f32),
                  jax.ShapeDtypeStruct((E, D // 2), f32)),
        mesh=_MESH,
        scratch_types=[
            pltpu.VMEM((CH,), jnp.int32),
            pltpu.VMEM((CH,), jnp.int32),
            pltpu.VMEM((CH, D), f32),
            pltpu.VMEM((CH, D), f32),
            pltpu.VMEM((CH, D), f32),
            pltpu.VMEM((CH, D // 2), f32),
            pltpu.VMEM((CH, D // 2), f32),
            pltpu.SemaphoreType.DMA,
            pltpu.SemaphoreType.DMA,
            pltpu.SemaphoreType.DMA,
        ],
    )
    return gk(src, dst, dh, eh, ce)


def _sc_scatter_body(dst_hbm, src_hbm, eplo_hbm, ephi_hbm, bhlo_hbm, bhhi_hbm,
                     siglo_hbm, sighi_hbm, shlo_hbm, shhi_hbm,
                     didx, sidx, xbuf, sbuf, bbuf, zbuf, sem, acc,
                     *, NP, E, CH):
    cid = lax.axis_index("c")
    sid = lax.axis_index("s")
    H = 128
    nchunks = E // CH
    per = (nchunks + NS - 1) // NS
    rows_pt = NP // NS
    zrows = 8
    nz = rows_pt // zrows

    def zrow(r, cc):
        for q in range(H // 16):
            zbuf[r, pl.ds(q * 16, 16)] = jnp.zeros((16,), f32)
        return cc

    lax.fori_loop(0, zrows, zrow, 0)

    def zero_acc():
        def zc(q, cc):
            pltpu.sync_copy(zbuf, acc.at[pl.ds(sid * rows_pt + q * zrows, zrows)])
            return cc
        lax.fori_loop(0, nz, zc, 0)

    def phase(ep_ref, bh_ref, mul_b):
        def chunk_body(j, carry):
            chunk = sid + j * NS

            @pl.when(chunk < nchunks)
            def _():
                base = chunk * CH
                pltpu.sync_copy(dst_hbm.at[pl.ds(base, CH)], didx)
                pltpu.sync_copy(ep_ref.at[pl.ds(base, CH)], xbuf)
                if mul_b:
                    pltpu.sync_copy(src_hbm.at[pl.ds(base, CH)], sidx)
                    pltpu.async_copy(bh_ref.at[sidx], bbuf, sem).wait()

                def row(r, cc):
                    for q in range(H // 16):
                        sl = pl.ds(q * 16, 16)
                        x = xbuf[r, sl]
                        s = 1.0 / (1.0 + jnp.exp(-x))
                        if mul_b:
                            s = s * bbuf[r, sl]
                        sbuf[r, sl] = s
                    return cc

                lax.fori_loop(0, CH, row, 0)
                pltpu.sync_copy(sbuf, acc.at[didx], add=True)

            return carry

        lax.fori_loop(0, per, chunk_body, 0)

    def dump(out_ref):
        pltpu.sync_copy(acc.at[pl.ds(sid * rows_pt, rows_pt)],
                        out_ref.at[pl.ds(sid * rows_pt, rows_pt)])

    def run(ep_ref, bh_ref, sig_ref, sh_ref):
        zero_acc()
        plsc.subcore_barrier()
        phase(ep_ref, bh_ref, False)
        plsc.subcore_barrier()
        dump(sig_ref)
        zero_acc()
        plsc.subcore_barrier()
        phase(ep_ref, bh_ref, True)
        plsc.subcore_barrier()
        dump(sh_ref)

    @pl.when(cid == 0)
    def _():
        run(eplo_hbm, bhlo_hbm, siglo_hbm, shlo_hbm)

    @pl.when(cid == 1)
    def _():
        run(ephi_hbm, bhhi_hbm, sighi_hbm, shhi_hbm)


def _sc_scatter(dst, src, ep_lo, ep_hi, bh_lo, bh_hi):
    E = dst.shape[0]
    N = bh_lo.shape[0]
    unit = NS * 8
    NP = ((N + unit - 1) // unit) * unit
    H = 128
    CH = 64
    sk = pl.kernel(
        functools.partial(_sc_scatter_body, NP=NP, E=E, CH=CH),
        out_type=(jax.ShapeDtypeStruct((NP, H), f32),
                  jax.ShapeDtypeStruct((NP, H), f32),
                  jax.ShapeDtypeStruct((NP, H), f32),
                  jax.ShapeDtypeStruct((NP, H), f32)),
        mesh=_MESH,
        scratch_types=[
            pltpu.VMEM((CH,), jnp.int32),
            pltpu.VMEM((CH,), jnp.int32),
            pltpu.VMEM((CH, H), f32),
            pltpu.VMEM((CH, H), f32),
            pltpu.VMEM((CH, H), f32),
            pltpu.VMEM((8, H), f32),
            pltpu.SemaphoreType.DMA,
            pltpu.VMEM_SHARED((NP, H), f32),
        ],
    )
    a, b, c, d = sk(dst, src, ep_lo, ep_hi, bh_lo, bh_hi)
    return a[:N], b[:N], c[:N], d[:N]


def _fin_h_a_body(ah_ref, slo_ref, shi_ref, hlo_ref, hhi_ref, hnew_ref, st_ref):
    i = pl.program_id(0)
    ssig = jnp.concatenate([slo_ref[...], shi_ref[...]], axis=1)
    ssh = jnp.concatenate([hlo_ref[...], hhi_ref[...]], axis=1)
    hn = ah_ref[...] + ssh / (ssig + 1e-6)
    hnew_ref[...] = hn
    s1 = jnp.sum(hn, axis=0, keepdims=True)
    s2 = jnp.sum(hn * hn, axis=0, keepdims=True)
    blk = jnp.concatenate([s1, s2], axis=0)

    @pl.when(i == 0)
    def _():
        st_ref[...] = blk

    @pl.when(i > 0)
    def _():
        st_ref[...] = st_ref[...] + blk


def _fin_h_a(ah, ssig_lo, ssig_hi, ssh_lo, ssh_hi):
    N, D = ah.shape
    BN = 2000
    row = pl.BlockSpec((BN, D), lambda i: (i, 0))
    half = pl.BlockSpec((BN, D // 2), lambda i: (i, 0))
    return pl.pallas_call(
        _fin_h_a_body,
        grid=(N // BN,),
        in_specs=[row, half, half, half, half],
        out_specs=[row, pl.BlockSpec((2, D), lambda i: (0, 0))],
        out_shape=[jax.ShapeDtypeStruct((N, D), f32),
                   jax.ShapeDtypeStruct((2, D), f32)],
    )(ah, ssig_lo, ssig_hi, ssh_lo, ssh_hi)


def _fin_norm_body(x_ref, xn_ref, st_ref, g_ref, b_ref, out_ref, *, count):
    mean = st_ref[0:1, :] / count
    var = st_ref[1:2, :] / count - mean * mean
    inv = lax.rsqrt(var + 1e-5)
    xn = (xn_ref[...] - mean) * inv * g_ref[...] + b_ref[...]
    out_ref[...] = x_ref[...] + jnp.maximum(xn, 0.0)


def _fin_h_b(h, h_new, st, gamma, beta):
    N, D = h.shape
    BN = 2000
    row = pl.BlockSpec((BN, D), lambda i: (i, 0))
    return pl.pallas_call(
        functools.partial(_fin_norm_body, count=float(N)),
        grid=(N // BN,),
        in_specs=[row, row, pl.BlockSpec((2, D), lambda i: (0, 0)),
                  pl.BlockSpec((1, D), lambda i: (0, 0)),
                  pl.BlockSpec((1, D), lambda i: (0, 0))],
        out_specs=row,
        out_shape=jax.ShapeDtypeStruct((N, D), f32),
    )(h, h_new, st, gamma.reshape(1, D), beta.reshape(1, D))


def _fin_e_a_body(lo_ref, hi_ref, st_ref):
    i = pl.program_id(0)
    x = jnp.concatenate([lo_ref[...], hi_ref[...]], axis=1)
    s1 = jnp.sum(x, axis=0, keepdims=True)
    s2 = jnp.sum(x * x, axis=0, keepdims=True)
    blk = jnp.concatenate([s1, s2], axis=0)

    @pl.when(i == 0)
    def _():
        st_ref[...] = blk

    @pl.when(i > 0)
    def _():
        st_ref[...] = st_ref[...] + blk


def _fin_e_a(ep_lo, ep_hi):
    E, H = ep_lo.shape
    D = 2 * H
    BE = 2000
    half = pl.BlockSpec((BE, H), lambda i: (i, 0))
    return pl.pallas_call(
        _fin_e_a_body,
        grid=(E // BE,),
        in_specs=[half, half],
        out_specs=pl.BlockSpec((2, D), lambda i: (0, 0)),
        out_shape=jax.ShapeDtypeStruct((2, D), f32),
    )(ep_lo, ep_hi)


def _fin_e_b_body(e_ref, lo_ref, hi_ref, st_ref, g_ref, b_ref, out_ref, *, count):
    mean = st_ref[0:1, :] / count
    var = st_ref[1:2, :] / count - mean * mean
    inv = lax.rsqrt(var + 1e-5)
    x = jnp.concatenate([lo_ref[...], hi_ref[...]], axis=1)
    xn = (x - mean) * inv * g_ref[...] + b_ref[...]
    out_ref[...] = e_ref[...] + jnp.maximum(xn, 0.0)


def _fin_e_b(e, ep_lo, ep_hi, st, gamma, beta):
    E, D = e.shape
    BE = 2000
    row = pl.BlockSpec((BE, D), lambda i: (i, 0))
    half = pl.BlockSpec((BE, D // 2), lambda i: (i, 0))
    return pl.pallas_call(
        functools.partial(_fin_e_b_body, count=float(E)),
        grid=(E // BE,),
        in_specs=[row, half, half, pl.BlockSpec((2, D), lambda i: (0, 0)),
                  pl.BlockSpec((1, D), lambda i: (0, 0)),
                  pl.BlockSpec((1, D), lambda i: (0, 0))],
        out_specs=row,
        out_shape=jax.ShapeDtypeStruct((E, D), f32),
    )(e, ep_lo, ep_hi, st, gamma.reshape(1, D), beta.reshape(1, D))


def kernel(h, e, edge_index, WA, bA, WB, bB, WC, bC, WD, bD, WEm, bEm,
           bn_gh, bn_bh, bn_ge, bn_be):
    src = edge_index[0]
    dst = edge_index[1]
    ah, bh_lo, bh_hi, dh, eh = _node_mms(h, WA, bA, WB, bB, WD, bD, WEm, bEm)
    ce = _edge_mm(e, WC, bC)
    ep_lo, ep_hi = _sc_gather(src, dst, dh, eh, ce)
    ssig_lo, ssig_hi, ssh_lo, ssh_hi = _sc_scatter(dst, src, ep_lo, ep_hi,
                                                   bh_lo, bh_hi)
    h_new, st_h = _fin_h_a(ah, ssig_lo, ssig_hi, ssh_lo, ssh_hi)
    h_out = _fin_h_b(h, h_new, st_h, bn_gh, bn_bh)
    st_e = _fin_e_a(ep_lo, ep_hi)
    e_out = _fin_e_b(e, ep_lo, ep_hi, st_e, bn_ge, bn_be)
    return (h_out, e_out)
